# Initial kernel scaffold; baseline (speedup 1.0000x reference)
import jax
import jax.numpy as jnp
from jax.experimental import pallas as pl


def kernel(op_code, op_feats, edge_index, batch_idx, emb, W_feat, ln_g, ln_b, W_pre, b_pre, Wl0, bl0, Wr0, Wl1, bl1, Wr1, Wl2, bl2, Wr2, Wh1, bh1, Wh2, bh2):
    raise NotImplementedError("write your pallas kernel here")


# speedup vs baseline: 2.0723x; 2.0723x over previous
import jax
import jax.numpy as jnp
from jax import lax
from jax.experimental import pallas as pl
from jax.experimental.pallas import tpu as pltpu
from jax.experimental.pallas import tpu_sc as plsc

N = 10000
HALF = 5000
HALF_PAD = 5120
NPAD = 2 * HALF_PAD
D = 256
HID = 128
E = 320000
CH = 128
NT = 16
NW = 32
TPR = NPAD // NW
ACC_R = TPR + 8
BLK = 1024
NBLK = NPAD // BLK
F32 = jnp.float32
I32 = jnp.int32


def _iota16():
    return lax.broadcasted_iota(I32, (16,), 0)


def _span(spans_v, t):
    return spans_v[pl.ds(t * 16, 16)][0]


def _edge_body_deg(ssrc_hbm, sdst_hbm, spans_hbm, y_hbm, zacc_hbm, zdeg_hbm,
                   acc_out, deg_out,
                   spans_v, sidx_v, dtmp_v, rows_v, acc_v, deg_v, sem):
    _edge_common(True, ssrc_hbm, sdst_hbm, spans_hbm, y_hbm, zacc_hbm,
                 zdeg_hbm, acc_out, deg_out, spans_v, sidx_v, dtmp_v, rows_v,
                 acc_v, deg_v, sem)


def _edge_body_nodeg(ssrc_hbm, sdst_hbm, spans_hbm, y_hbm, zacc_hbm,
                     acc_out,
                     spans_v, sidx_v, dtmp_v, rows_v, acc_v, sem):
    _edge_common(False, ssrc_hbm, sdst_hbm, spans_hbm, y_hbm, zacc_hbm, None,
                 acc_out, None, spans_v, sidx_v, dtmp_v, rows_v, acc_v, None,
                 sem)


def _edge_common(with_deg, ssrc_hbm, sdst_hbm, spans_hbm, y_hbm, zacc_hbm,
                 zdeg_hbm, acc_out, deg_out, spans_v, sidx_v, dtmp_v, rows_v,
                 acc_v, deg_v, sem):
    c = lax.axis_index("c")
    s = lax.axis_index("s")
    w = c * NT + s
    iota = _iota16()
    ones_v = jnp.ones((16,), F32)

    pltpu.sync_copy(spans_hbm, spans_v)
    pltpu.sync_copy(zacc_hbm, acc_v)
    if with_deg:
        pltpu.sync_copy(zdeg_hbm, deg_v)

    lo = _span(spans_v, w)
    hi = _span(spans_v, w + 1)
    start0 = (lo // CH) * CH
    nch = (hi - start0 + CH - 1) // CH
    rbase = w * TPR

    def chunk(i, carry):
        base = start0 + i * CH
        pltpu.sync_copy(ssrc_hbm.at[pl.ds(base, CH)], sidx_v)
        pltpu.sync_copy(sdst_hbm.at[pl.ds(base, CH)], dtmp_v)
        for g in range(CH // 16):
            sl = pl.ds(g * 16, 16)
            eidx = (base + g * 16) + iota
            m = (eidx >= lo) & (eidx < hi)
            dloc = dtmp_v[sl] - rbase
            dtmp_v[sl] = jnp.where(m, dloc, TPR)
            sidx_v[sl] = jnp.where(m, sidx_v[sl], 0)
        pltpu.async_copy(y_hbm.at[sidx_v], rows_v, sem).wait()

        def edge(e, carry2):
            db = plsc.load_gather(dtmp_v, [jnp.full((16,), e, I32)])
            if with_deg:
                plsc.addupdate_scatter(deg_v, [db * 16 + iota], ones_v)
            for j in range(D // 16):
                vals = rows_v[e, pl.ds(j * 16, 16)]
                plsc.addupdate_scatter(acc_v, [db, j * 16 + iota], vals)
            return carry2

        lax.fori_loop(0, CH, edge, 0)
        return carry

    lax.fori_loop(0, nch, chunk, 0)

    pltpu.sync_copy(acc_v.at[pl.ds(0, TPR)], acc_out.at[pl.ds(rbase, TPR)])
    if with_deg:
        pltpu.sync_copy(deg_v.at[pl.ds(0, TPR * 16)],
                        deg_out.at[pl.ds(rbase * 16, TPR * 16)])


def _make_edge_kernel(with_deg):
    mesh = plsc.VectorSubcoreMesh(core_axis_name="c", subcore_axis_name="s")
    if with_deg:
        out_type = (jax.ShapeDtypeStruct((NPAD, D), F32),
                    jax.ShapeDtypeStruct((NPAD * 16,), F32))
        scratch = [
            pltpu.VMEM((48 * 16,), I32),
            pltpu.VMEM((CH,), I32),
            pltpu.VMEM((CH,), I32),
            pltpu.VMEM((CH, D), F32),
            pltpu.VMEM((ACC_R, D), F32),
            pltpu.VMEM((ACC_R * 16,), F32),
            pltpu.SemaphoreType.DMA,
        ]
        return pl.kernel(_edge_body_deg, out_type=out_type, mesh=mesh,
                         scratch_types=scratch,
                         compiler_params=pltpu.CompilerParams(
                             needs_layout_passes=False))
    out_type = jax.ShapeDtypeStruct((NPAD, D), F32)
    scratch = [
        pltpu.VMEM((48 * 16,), I32),
        pltpu.VMEM((CH,), I32),
        pltpu.VMEM((CH,), I32),
        pltpu.VMEM((CH, D), F32),
        pltpu.VMEM((ACC_R, D), F32),
        pltpu.SemaphoreType.DMA,
    ]
    return pl.kernel(_edge_body_nodeg, out_type=out_type, mesh=mesh,
                     scratch_types=scratch,
                     compiler_params=pltpu.CompilerParams(
                         needs_layout_passes=False))


def _encoder_body(opc_ref, feats_ref, emb_ref, wf_ref, lng_ref, lnb_ref,
                  wpre_ref, bpre_ref, x_ref):
    opc = opc_ref[0, 0, :]
    onehot = (opc[:, None] == lax.broadcasted_iota(I32, (BLK, HID), 1))
    e = jnp.dot(onehot.astype(F32), emb_ref[...], preferred_element_type=F32)
    xf = jnp.dot(feats_ref[...], wf_ref[...], preferred_element_type=F32)
    x = e + xf
    mu = jnp.mean(x, axis=-1, keepdims=True)
    var = jnp.mean((x - mu) ** 2, axis=-1, keepdims=True)
    x = (x - mu) * lax.rsqrt(var + 1e-12) * lng_ref[...] + lnb_ref[...]
    x_ref[...] = jnp.maximum(
        jnp.dot(x, wpre_ref[...], preferred_element_type=F32)
        + bpre_ref[...], 0.0)


def _layer_body(acc_ref, deg_ref, x_ref, wl_ref, bl_ref, wr_ref, xn_ref):
    deg = jnp.maximum(deg_ref[:, 0:1], 1.0)
    agg = acc_ref[...] / deg
    al = jnp.dot(agg, wl_ref[...], preferred_element_type=F32)
    r = jnp.dot(x_ref[...], wr_ref[...], preferred_element_type=F32)
    xn_ref[...] = jnp.maximum(al + bl_ref[...] + r, 0.0)


def _pool_body(x_ref, bi_ref, wh1_ref, bh1_ref, wh2_ref, bh2_ref, out_ref,
               sum_s, max_s, cnt_s):
    i = pl.program_id(0)

    @pl.when(i == 0)
    def _():
        sum_s[...] = jnp.zeros((16, D), F32)
        max_s[...] = jnp.full((16, D), -jnp.inf, F32)
        cnt_s[...] = jnp.zeros((16, 128), F32)

    x = x_ref[...]
    bi = bi_ref[0, 0, :]
    onehot = (bi[:, None] == lax.broadcasted_iota(I32, (BLK, 16), 1)
              ).astype(F32)
    sum_s[...] += lax.dot_general(onehot, x, (((0,), (0,)), ((), ())),
                                  preferred_element_type=F32)
    cnt_s[...] += jnp.sum(onehot, axis=0)[:, None]
    parts = []
    for g in range(16):
        parts.append(jnp.max(jnp.where(bi[:, None] == g, x, -jnp.inf),
                             axis=0, keepdims=True))
    max_s[...] = jnp.maximum(max_s[...], jnp.concatenate(parts, axis=0))

    @pl.when(i == NBLK - 1)
    def _():
        cnt = jnp.maximum(cnt_s[:, 0:1], 1.0)
        be = max_s[...] + sum_s[...] / cnt
        nrm = jnp.sqrt(jnp.sum(be * be, axis=-1, keepdims=True))
        ge = be / nrm
        h = jnp.maximum(jnp.dot(ge, wh1_ref[...], preferred_element_type=F32)
                        + bh1_ref[...], 0.0)
        out_ref[...] = (jnp.dot(h, wh2_ref[...], preferred_element_type=F32)
                        + bh2_ref[...])


def _encoder_call(opc3, feats, emb_pad, wf, lng, lnb, wpre, bpre):
    return pl.pallas_call(
        _encoder_body,
        grid=(NBLK,),
        in_specs=[
            pl.BlockSpec((1, 1, BLK), lambda i: (i, 0, 0)),
            pl.BlockSpec((BLK, 140), lambda i: (i, 0)),
            pl.BlockSpec((HID, HID), lambda i: (0, 0)),
            pl.BlockSpec((140, HID), lambda i: (0, 0)),
            pl.BlockSpec((1, HID), lambda i: (0, 0)),
            pl.BlockSpec((1, HID), lambda i: (0, 0)),
            pl.BlockSpec((HID, D), lambda i: (0, 0)),
            pl.BlockSpec((1, D), lambda i: (0, 0)),
        ],
        out_specs=pl.BlockSpec((BLK, D), lambda i: (i, 0)),
        out_shape=jax.ShapeDtypeStruct((NPAD, D), F32),
    )(opc3, feats, emb_pad, wf, lng, lnb, wpre, bpre)


def _layer_call(acc, deg, x, wl, bl, wr):
    specs = [
        pl.BlockSpec((BLK, D), lambda i: (i, 0)),
        pl.BlockSpec((BLK, 16), lambda i: (i, 0)),
        pl.BlockSpec((BLK, D), lambda i: (i, 0)),
        pl.BlockSpec((D, D), lambda i: (0, 0)),
        pl.BlockSpec((1, D), lambda i: (0, 0)),
        pl.BlockSpec((D, D), lambda i: (0, 0)),
    ]
    return pl.pallas_call(
        _layer_body,
        grid=(NBLK,),
        in_specs=specs,
        out_specs=pl.BlockSpec((BLK, D), lambda i: (i, 0)),
        out_shape=jax.ShapeDtypeStruct((NPAD, D), F32),
    )(acc, deg, x, wl, bl, wr)


def _pool_call(x3, bi3, wh1, bh1, wh2, bh2):
    return pl.pallas_call(
        _pool_body,
        grid=(NBLK,),
        in_specs=[
            pl.BlockSpec((BLK, D), lambda i: (i, 0)),
            pl.BlockSpec((1, 1, BLK), lambda i: (i, 0, 0)),
            pl.BlockSpec((D, D), lambda i: (0, 0)),
            pl.BlockSpec((1, D), lambda i: (0, 0)),
            pl.BlockSpec((D, 1), lambda i: (0, 0)),
            pl.BlockSpec((1, 1), lambda i: (0, 0)),
        ],
        out_specs=pl.BlockSpec((16, 1), lambda i: (0, 0)),
        out_shape=jax.ShapeDtypeStruct((16, 1), F32),
        scratch_shapes=[pltpu.VMEM((16, D), F32),
                        pltpu.VMEM((16, D), F32),
                        pltpu.VMEM((16, 128), F32)],
    )(x3, bi3, wh1, bh1, wh2, bh2)


def kernel(op_code, op_feats, edge_index, batch_idx, emb, W_feat, ln_g, ln_b,
           W_pre, b_pre, Wl0, bl0, Wr0, Wl1, bl1, Wr1, Wl2, bl2, Wr2,
           Wh1, bh1, Wh2, bh2):
    pad_i = jnp.full((HALF_PAD - HALF,), 120, I32)
    pad_b = jnp.full((HALF_PAD - HALF,), 16, I32)
    pad_f = jnp.zeros((HALF_PAD - HALF, 140), F32)
    opc_pad = jnp.concatenate([op_code[:HALF], pad_i, op_code[HALF:], pad_i])
    feats_pad = jnp.concatenate([op_feats[:HALF], pad_f,
                                 op_feats[HALF:], pad_f])
    bi_pad = jnp.concatenate([batch_idx[:HALF], pad_b, batch_idx[HALF:], pad_b])
    emb_pad = jnp.zeros((HID, HID), F32).at[:emb.shape[0]].set(emb)

    opc3 = opc_pad.reshape(NBLK, 1, BLK)
    bi3 = bi_pad.reshape(NBLK, 1, BLK)

    src = edge_index[0]
    dst = edge_index[1]
    spad = src + jnp.where(src >= HALF, HALF_PAD - HALF, 0)
    dpad = dst + jnp.where(dst >= HALF, HALF_PAD - HALF, 0)
    order = jnp.argsort(dpad)
    ssrc = jnp.concatenate([spad[order], jnp.zeros((CH,), I32)])
    sdst = jnp.concatenate([dpad[order], jnp.zeros((CH,), I32)])
    spans = jnp.searchsorted(sdst[:E], jnp.arange(33, dtype=I32) * TPR
                             ).astype(I32)
    spans = jnp.concatenate([spans, jnp.full((15,), E, I32)])
    spans = jnp.repeat(spans, 16)

    zacc = jnp.zeros((ACC_R, D), F32)
    zdeg = jnp.zeros((ACC_R * 16,), F32)

    lng = ln_g.reshape(1, HID)
    lnb = ln_b.reshape(1, HID)
    bpre = b_pre.reshape(1, D)
    bh1r = bh1.reshape(1, D)
    bh2r = bh2.reshape(1, 1)

    x0 = _encoder_call(opc3, feats_pad, emb_pad, W_feat, lng, lnb,
                       W_pre, bpre)

    edge_deg = _make_edge_kernel(True)
    edge_nodeg = _make_edge_kernel(False)

    acc0, deg = edge_deg(ssrc, sdst, spans, x0, zacc, zdeg)
    deg = deg.reshape(NPAD, 16)
    x1 = _layer_call(acc0, deg, x0, Wl0, bl0.reshape(1, D), Wr0)
    acc1 = edge_nodeg(ssrc, sdst, spans, x1, zacc)
    x2 = _layer_call(acc1, deg, x1, Wl1, bl1.reshape(1, D), Wr1)
    acc2 = edge_nodeg(ssrc, sdst, spans, x2, zacc)
    x3 = _layer_call(acc2, deg, x2, Wl2, bl2.reshape(1, D), Wr2)

    return _pool_call(x3, bi3, Wh1, bh1r, Wh2, bh2r)
